# baseline (speedup 1.0000x reference)
import functools

import jax
import jax.numpy as jnp
from jax import lax
from jax.experimental import pallas as pl
from jax.experimental.pallas import tpu as pltpu
from jax.experimental.pallas import tpu_sc as plsc

NUM_EMB = 1000000
D = 64
DP = 128
B_TOK = 4096
T_TOK = 200
TP = 256
NC = 2
NS = 16
NW = NC * NS
BPW = B_TOK // NW
CHUNK = 128
REM = T_TOK - CHUNK


def _sc_gather(table, idx3):
    mesh = plsc.VectorSubcoreMesh(core_axis_name="c", subcore_axis_name="s")

    @functools.partial(
        pl.kernel,
        mesh=mesh,
        out_type=jax.ShapeDtypeStruct((B_TOK, T_TOK, DP), jnp.float32),
        compiler_params=pltpu.CompilerParams(use_tc_tiling_on_sc=True),
        scratch_types=[
            pltpu.VMEM((BPW, TP), jnp.int32),
            pltpu.VMEM((2 * T_TOK, DP), jnp.float32),
            pltpu.SemaphoreType.DMA,
            pltpu.SemaphoreType.DMA,
        ],
    )
    def k(table_hbm, idx_hbm, out_hbm, idx_v, rows_v, sem_in, sem_out):
        wid = lax.axis_index("s") * NC + lax.axis_index("c")
        pltpu.sync_copy(idx_hbm.at[wid], idx_v)

        def gathers(j, half):
            return (
                pltpu.make_async_copy(
                    table_hbm.at[idx_v.at[j, pl.ds(0, CHUNK)]],
                    rows_v.at[pl.ds(half * T_TOK, CHUNK)],
                    sem_in,
                ),
                pltpu.make_async_copy(
                    table_hbm.at[idx_v.at[j, pl.ds(CHUNK, REM)]],
                    rows_v.at[pl.ds(half * T_TOK + CHUNK, REM)],
                    sem_in,
                ),
            )

        def out_copy(j, half):
            return pltpu.make_async_copy(
                rows_v.at[pl.ds(half * T_TOK, T_TOK)],
                out_hbm.at[wid * BPW + j],
                sem_out,
            )

        for c in gathers(0, 0):
            c.start()

        def body(j, carry):
            half = lax.rem(j, 2)

            @pl.when(j >= 1)
            def _():
                out_copy(j - 1, 1 - half).wait()

            @pl.when(j + 1 < BPW)
            def _():
                for c in gathers(j + 1, 1 - half):
                    c.start()

            for c in gathers(j, half):
                c.wait()
            out_copy(j, half).start()
            return carry

        lax.fori_loop(0, BPW, body, 0)
        out_copy(BPW - 1, (BPW - 1) % 2).wait()

    return k(table, idx3)


def kernel(token_ids, embeddings):
    table = jnp.concatenate([embeddings, embeddings], axis=1)
    idx3 = jnp.pad(token_ids.reshape(NW, BPW, T_TOK), ((0, 0), (0, 0), (0, TP - T_TOK)))
    out = _sc_gather(table, idx3)
    return out[:, :, :D]


# speedup vs baseline: 1.0167x; 1.0167x over previous
import functools

import jax
import jax.numpy as jnp
from jax import lax
from jax.experimental import pallas as pl
from jax.experimental.pallas import tpu as pltpu
from jax.experimental.pallas import tpu_sc as plsc

NUM_EMB = 1000000
D = 64
DP = 128
B_TOK = 4096
T_TOK = 200
TP = 256
NC = 2
NS = 16
NW = NC * NS
BPW = B_TOK // NW
CHUNK = 128
REM = T_TOK - CHUNK


def _sc_gather(table, idx3):
    mesh = plsc.VectorSubcoreMesh(core_axis_name="c", subcore_axis_name="s")

    @functools.partial(
        pl.kernel,
        mesh=mesh,
        out_type=jax.ShapeDtypeStruct((B_TOK, T_TOK, DP), jnp.float32),
        compiler_params=pltpu.CompilerParams(use_tc_tiling_on_sc=True),
        scratch_types=[
            pltpu.VMEM((BPW, TP), jnp.int32),
            pltpu.VMEM((2 * T_TOK, DP), jnp.float32),
            pltpu.SemaphoreType.DMA,
            pltpu.SemaphoreType.DMA,
        ],
    )
    def k(table_hbm, idx_hbm, out_hbm, idx_v, rows_v, sem_in, sem_out):
        wid = lax.axis_index("s") * NC + lax.axis_index("c")
        pltpu.sync_copy(idx_hbm.at[wid], idx_v)

        def gathers(j, half):
            return (
                pltpu.make_async_copy(
                    table_hbm.at[idx_v.at[j, pl.ds(0, CHUNK)]],
                    rows_v.at[pl.ds(half * T_TOK, CHUNK)],
                    sem_in,
                ),
                pltpu.make_async_copy(
                    table_hbm.at[idx_v.at[j, pl.ds(CHUNK, REM)]],
                    rows_v.at[pl.ds(half * T_TOK + CHUNK, REM)],
                    sem_in,
                ),
            )

        def out_copy(j, half):
            return pltpu.make_async_copy(
                rows_v.at[pl.ds(half * T_TOK, T_TOK)],
                out_hbm.at[wid * BPW + j],
                sem_out,
            )

        for c in gathers(0, 0):
            c.start()

        def body(j, carry):
            half = lax.rem(j, 2)

            @pl.when(j >= 1)
            def _():
                out_copy(j - 1, 1 - half).wait()

            @pl.when(j + 1 < BPW)
            def _():
                for c in gathers(j + 1, 1 - half):
                    c.start()

            for c in gathers(j, half):
                c.wait()
            out_copy(j, half).start()
            return carry

        lax.fori_loop(0, BPW, body, 0)
        out_copy(BPW - 1, (BPW - 1) % 2).wait()

    return k(table, idx3)


def _pad_table(emb):
    pbk = 8000

    def body(in_ref, out_ref):
        out_ref[:, :D] = in_ref[:, :]
        out_ref[:, D:] = jnp.zeros((pbk, DP - D), jnp.float32)

    return pl.pallas_call(
        body,
        grid=(NUM_EMB // pbk,),
        in_specs=[pl.BlockSpec((pbk, D), lambda i: (i, 0))],
        out_specs=pl.BlockSpec((pbk, DP), lambda i: (i, 0)),
        out_shape=jax.ShapeDtypeStruct((NUM_EMB, DP), jnp.float32),
    )(emb)


def kernel(token_ids, embeddings):
    table = _pad_table(embeddings)
    idx3 = jnp.pad(token_ids.reshape(NW, BPW, T_TOK), ((0, 0), (0, 0), (0, TP - T_TOK)))
    out = _sc_gather(table, idx3)
    return out[:, :, :D]
